# baseline (speedup 1.0000x reference)
import jax
import jax.numpy as jnp
from jax.experimental import pallas as pl


def _g_kernel(w_ref, a_ref, b_ref, out_ref):
    a = a_ref[...] * w_ref[...]
    out_ref[...] = jnp.dot(a, b_ref[...], preferred_element_type=jnp.float32)


def kernel(DV2_H, invDE_HT_DV2, W):
    N, E = DV2_H.shape
    TM = 1024
    w2d = W.reshape(1, E)
    return pl.pallas_call(
        _g_kernel,
        grid=(N // TM,),
        in_specs=[
            pl.BlockSpec((1, E), lambda i: (0, 0)),
            pl.BlockSpec((TM, E), lambda i: (i, 0)),
            pl.BlockSpec((E, N), lambda i: (0, 0)),
        ],
        out_specs=pl.BlockSpec((TM, N), lambda i: (i, 0)),
        out_shape=jax.ShapeDtypeStruct((N, N), jnp.float32),
    )(w2d, DV2_H, invDE_HT_DV2)


# speedup vs baseline: 1.0685x; 1.0685x over previous
import jax
import jax.numpy as jnp
from jax.experimental import pallas as pl


def _g_kernel(w_ref, a_ref, b_ref, out_ref):
    a = a_ref[...] * w_ref[...]
    out_ref[...] = jnp.dot(a, b_ref[...], preferred_element_type=jnp.float32)


def kernel(DV2_H, invDE_HT_DV2, W):
    N, E = DV2_H.shape
    TM = 512
    w2d = W.reshape(1, E)
    return pl.pallas_call(
        _g_kernel,
        grid=(N // TM,),
        in_specs=[
            pl.BlockSpec((1, E), lambda i: (0, 0)),
            pl.BlockSpec((TM, E), lambda i: (i, 0)),
            pl.BlockSpec((E, N), lambda i: (0, 0)),
        ],
        out_specs=pl.BlockSpec((TM, N), lambda i: (i, 0)),
        out_shape=jax.ShapeDtypeStruct((N, N), jnp.float32),
    )(w2d, DV2_H, invDE_HT_DV2)


# speedup vs baseline: 1.0749x; 1.0060x over previous
import jax
import jax.numpy as jnp
from jax.experimental import pallas as pl
from jax.experimental.pallas import tpu as pltpu


def _g_kernel(w_ref, a_ref, b_ref, out_ref):
    a = a_ref[...] * w_ref[...]
    out_ref[...] = jnp.dot(a, b_ref[...], preferred_element_type=jnp.float32)


def kernel(DV2_H, invDE_HT_DV2, W):
    N, E = DV2_H.shape
    TM = 512
    w2d = W.reshape(1, E)
    return pl.pallas_call(
        _g_kernel,
        grid=(N // TM,),
        in_specs=[
            pl.BlockSpec((1, E), lambda i: (0, 0)),
            pl.BlockSpec((TM, E), lambda i: (i, 0)),
            pl.BlockSpec((E, N), lambda i: (0, 0)),
        ],
        out_specs=pl.BlockSpec((TM, N), lambda i: (i, 0)),
        out_shape=jax.ShapeDtypeStruct((N, N), jnp.float32),
        compiler_params=pltpu.CompilerParams(
            dimension_semantics=("parallel",),
        ),
    )(w2d, DV2_H, invDE_HT_DV2)
